# baseline (speedup 1.0000x reference)
import functools
import math

import jax
import jax.numpy as jnp
from jax.experimental import pallas as pl
from jax.experimental.pallas import tpu as pltpu

F32 = jnp.float32
BF16 = jnp.bfloat16

_L_INV = 1.0 / 3.0
_EPS = 1e-5
_TAU_INV = 2.0
_DECAY = 1e-4
_CL_W = 0.1
_DEG = 8


def _dot_t(a, x):
    return jax.lax.dot_general(
        a, x, (((0,), (0,)), ((), ())), preferred_element_type=F32)


def _simmat(x, y):
    return jax.lax.dot_general(
        x, y, (((1,), (1,)), ((), ())), preferred_element_type=F32)


def _body(a1_ref, a2_ref, t0_ref, te_ref, ce_ref, a3_ref,
          idx_ref, g_ref, b_ref, w1_ref, cb1_ref, w2_ref, cb2_ref,
          rel_ref, fcw_ref, fcb_ref,
          out_ref, aux_ref,
          u1_ref, u2_ref, st1_ref, st2_ref, sel_ref, ac1_ref, ac2_ref,
          relx_ref, oht_ref, ohc_ref,
          *, nb, blk, ne, nt, nc, nr, ns, e_cnt):
    step = pl.program_id(0)

    @pl.when(step == 0)
    def _():
        u1_ref[...] = jnp.zeros_like(u1_ref)
        u2_ref[...] = jnp.zeros_like(u2_ref)
        sel_ref[...] = jnp.zeros_like(sel_ref)
        et = idx_ref[ns + e_cnt:ns + 2 * e_cnt, :]
        lane_r = jax.lax.broadcasted_iota(jnp.int32, (e_cnt, nr), 1)
        rmod = et - nr * (et // nr)
        sgn = jnp.where(et >= nr, -1.0, 1.0)
        ohr = jnp.where(lane_r == rmod, sgn, 0.0).astype(BF16)
        relx_ref[...] = jnp.dot(ohr, rel_ref[...].astype(BF16),
                                preferred_element_type=F32)
        ids = idx_ref[0:ns, :]
        iota_t = jax.lax.broadcasted_iota(jnp.int32, (ns, nt), 1)
        iota_c = jax.lax.broadcasted_iota(jnp.int32, (ns, nc), 1)
        oht_ref[...] = ((iota_t == ids - ne) & (ids >= ne)
                        & (ids < ne + nt)).astype(BF16)
        ohc_ref[...] = ((iota_c == ids - (ne + nt))
                        & (ids >= ne + nt)).astype(BF16)

    @pl.when(step < nb)
    def _sweep():
        a1 = a1_ref[...].astype(BF16)
        a2 = a2_ref[...].astype(BF16)
        t0f = t0_ref[...]
        t0 = t0f.astype(BF16)
        t1a = jnp.dot(a1, te_ref[...].astype(BF16),
                      preferred_element_type=F32)
        t1b = jnp.dot(a2, ce_ref[...].astype(BF16),
                      preferred_element_type=F32)
        r1 = jnp.concatenate([t0, t1a.astype(BF16)], axis=1)
        r2 = jnp.concatenate([t0, t1b.astype(BF16)], axis=1)
        u1_ref[...] += _dot_t(a1, r1)
        u2_ref[...] += _dot_t(a2, r2)
        base = pl.multiple_of(step * blk, blk)
        st1_ref[pl.ds(base, blk), :] = ((t0f + t1a)
                                        * _L_INV).astype(BF16)
        st2_ref[pl.ds(base, blk), :] = ((t0f + t1b)
                                        * _L_INV).astype(BF16)
        ac1_ref[pl.ds(base, blk), :] = a1
        ac2_ref[pl.ds(base, blk), :] = a2

    nb2 = max(1, nb // 2)
    blk2 = (nb * blk) // nb2

    @pl.when((step >= nb) & (step < nb + nb2))
    def _select():
        d = te_ref.shape[1]
        j = step - nb
        base = pl.multiple_of(j * blk2, blk2)
        v1 = (u1_ref[:, :d] * _L_INV).astype(BF16)
        v2 = (u2_ref[:, :d] * _L_INV).astype(BF16)
        a1 = ac1_ref[pl.ds(base, blk2), :]
        a2 = ac2_ref[pl.ds(base, blk2), :]
        pre1 = (st1_ref[pl.ds(base, blk2), :].astype(F32)
                + jnp.dot(a1, v1, preferred_element_type=F32))
        pre2 = (st2_ref[pl.ds(base, blk2), :].astype(F32)
                + jnp.dot(a2, v2, preferred_element_type=F32))
        ids = idx_ref[0:ns, :]
        iota_b = jax.lax.broadcasted_iota(jnp.int32, (ns, blk2), 1)
        oh = (iota_b == ids - base).astype(BF16)
        pre = jnp.concatenate([pre1, pre2], axis=1).astype(BF16)
        sel_ref[...] += jnp.dot(oh, pre, preferred_element_type=F32)

    @pl.when(step == nb + nb2)
    def _finish():
        g = g_ref[...]
        b = b_ref[...]

        def ln(x):
            mu = jnp.mean(x, axis=-1, keepdims=True)
            var = jnp.mean((x - mu) * (x - mu), axis=-1, keepdims=True)
            return (x - mu) * jax.lax.rsqrt(var + _EPS) * g + b

        te = te_ref[...]
        ce = ce_ref[...]
        d = te.shape[1]

        a3 = a3_ref[...].astype(BF16)
        b1_3 = _dot_t(a3, te.astype(BF16))
        t1_3 = jnp.dot(a3, ce.astype(BF16), preferred_element_type=F32)
        b2_3 = _dot_t(a3, t1_3.astype(BF16))
        tsum_3 = jnp.dot(a3, (ce + b1_3).astype(BF16),
                         preferred_element_type=F32)
        t2c_t = ln((te + tsum_3) * _L_INV)
        t2c_c = ln((ce + b1_3 + b2_3) * _L_INV)

        u1 = u1_ref[...]
        u2 = u2_ref[...]
        e2t_t = ln((te + u1[:, :d] + u1[:, d:]) * _L_INV)
        e2c_c = ln((ce + u2[:, :d] + u2[:, d:]) * _L_INV)

        sel = sel_ref[...]
        top1 = ln(sel[:, :d])
        top2 = ln(sel[:, d:])

        ids = idx_ref[0:ns, :]
        is_ent = ids < ne
        oh_t = oht_ref[...]
        oh_c = ohc_ref[...]
        small1 = jnp.concatenate([e2t_t, t2c_t], axis=1).astype(BF16)
        small2 = jnp.concatenate([t2c_c, e2c_c], axis=1).astype(BF16)
        both = (jnp.dot(oh_t, small1, preferred_element_type=F32)
                + jnp.dot(oh_c, small2, preferred_element_type=F32))
        s1 = jnp.where(is_ent, top1, 0.0) + both[:, :d]
        s2 = jnp.where(is_ent, top2, 0.0) + both[:, d:]

        def fc(x):
            h = (jnp.dot(x, w1_ref[...], preferred_element_type=F32)
                 + cb1_ref[...])
            h = jnp.where(h > 0.0, h, jnp.exp(jnp.minimum(h, 0.0)) - 1.0)
            return (jnp.dot(h, w2_ref[...], preferred_element_type=F32)
                    + cb2_ref[...])

        def normalize(z):
            nrm = jnp.sqrt(jnp.sum(z * z, axis=-1, keepdims=True))
            return z / jnp.maximum(nrm, 1e-12)

        av = normalize(fc(s1))
        bv = normalize(fc(s2))
        ab = av.astype(BF16)
        bb = bv.astype(BF16)
        self_sim = math.exp(_TAU_INV)
        r1m = jnp.exp(_simmat(ab, ab) * _TAU_INV)
        btm = jnp.exp(_simmat(ab, bb) * _TAU_INV)
        r2m = jnp.exp(_simmat(bb, bb) * _TAU_INV)
        pos = jnp.sum(av * bv, axis=-1, keepdims=True) * _TAU_INV
        denom1 = (jnp.sum(r1m, axis=1, keepdims=True)
                  + jnp.sum(btm, axis=1, keepdims=True) - self_sim)
        denom2 = (jnp.sum(r2m, axis=0, keepdims=True)
                  + jnp.sum(btm, axis=0, keepdims=True) - self_sim)
        total = (jnp.sum(jnp.log(denom1)) + jnp.sum(jnp.log(denom2))
                 - 2.0 * jnp.sum(pos))
        contrast = total * (0.5 / float(ns))

        rel = relx_ref[...]

        es = idx_ref[ns:ns + e_cnt, :]
        lane_n = jax.lax.broadcasted_iota(jnp.int32, (e_cnt, ns), 1)
        ohs = (lane_n == es).astype(BF16)
        srcc = jnp.concatenate([s1, s2], axis=1).astype(BF16)
        sm = jnp.dot(ohs, srcc, preferred_element_type=F32)

        emb_reg = 0.5 * (jnp.sum(s1 * s1) + jnp.sum(s2 * s2)
                         + jnp.sum(rel * rel))
        emb_loss = _DECAY * emb_reg / float(e_cnt)
        aux = _CL_W * contrast + emb_loss

        msg = jnp.maximum(sm + rel, 0.0)
        two_d = msg.shape[1]
        pooled = jnp.mean(msg.reshape(e_cnt // _DEG, _DEG, two_d), axis=1)
        predict = (jnp.dot(pooled.astype(BF16), fcw_ref[...].astype(BF16),
                           preferred_element_type=F32) + fcb_ref[...])
        out_ref[...] = jax.nn.sigmoid(predict)
        aux_ref[...] = jnp.zeros(aux_ref.shape, F32) + aux


def kernel(entity_emb, type_emb, cluster_emb, relation, ln_gamma, ln_beta,
           cl_w1, cl_b1, cl_w2, cl_b2, fc_w, fc_b,
           g_e2t, g_t2c, g_e2c, src_ids, etype, edge_src):
    ne, d = entity_emb.shape
    nt = type_emb.shape[0]
    nc = cluster_emb.shape[0]
    nr = relation.shape[0]
    ns = src_ids.shape[0]
    e_cnt = etype.shape[0]
    n_types = fc_w.shape[1]
    blk = 2048 if ne % 2048 == 0 else ne
    nb = ne // blk

    body = functools.partial(_body, nb=nb, blk=blk, ne=ne, nt=nt, nc=nc,
                             nr=nr, ns=ns, e_cnt=e_cnt)
    idx = jnp.concatenate([src_ids.astype(jnp.int32),
                           edge_src.astype(jnp.int32),
                           etype.astype(jnp.int32)])[:, None]

    def adj_spec(w):
        return pl.BlockSpec((blk, w), lambda i: (jnp.minimum(i, nb - 1), 0))

    def t0_spec():
        return pl.BlockSpec((blk, d), lambda i: (jnp.minimum(i, nb - 1), 0))

    def const_spec(shape):
        n_ = len(shape)
        return pl.BlockSpec(shape, lambda i, _n=n_: (0,) * _n)

    out, aux = pl.pallas_call(
        body,
        grid=(nb + max(1, nb // 2) + 1,),
        in_specs=[
            adj_spec(nt), adj_spec(nc), t0_spec(),
            const_spec((nt, d)), const_spec((nc, d)), const_spec((nt, nc)),
            const_spec((ns + 2 * e_cnt, 1)),
            const_spec((1, d)), const_spec((1, d)),
            const_spec((d, d)), const_spec((1, d)),
            const_spec((d, d)), const_spec((1, d)),
            const_spec((nr, 2 * d)), const_spec((2 * d, n_types)),
            const_spec((1, n_types)),
        ],
        out_specs=(const_spec((e_cnt // _DEG, n_types)),
                   const_spec((1, 128))),
        out_shape=(jax.ShapeDtypeStruct((e_cnt // _DEG, n_types), F32),
                   jax.ShapeDtypeStruct((1, 128), F32)),
        scratch_shapes=[pltpu.VMEM((nt, 2 * d), F32),
                        pltpu.VMEM((nc, 2 * d), F32),
                        pltpu.VMEM((ne, d), BF16),
                        pltpu.VMEM((ne, d), BF16),
                        pltpu.VMEM((ns, 2 * d), F32),
                        pltpu.VMEM((ne, nt), BF16),
                        pltpu.VMEM((ne, nc), BF16),
                        pltpu.VMEM((e_cnt, 2 * d), F32),
                        pltpu.VMEM((ns, nt), BF16),
                        pltpu.VMEM((ns, nc), BF16)],
        compiler_params=pltpu.CompilerParams(
            dimension_semantics=("arbitrary",),
            vmem_limit_bytes=100 * 1024 * 1024,
        ),
    )(g_e2t, g_e2c, entity_emb, type_emb, cluster_emb, g_t2c,
      idx, ln_gamma, ln_beta,
      cl_w1, cl_b1, cl_w2, cl_b2,
      relation, fc_w, fc_b)

    return out, aux[0, 0]


# speedup vs baseline: 1.0367x; 1.0367x over previous
import functools
import math

import jax
import jax.numpy as jnp
from jax.experimental import pallas as pl
from jax.experimental.pallas import tpu as pltpu

F32 = jnp.float32
BF16 = jnp.bfloat16

_L_INV = 1.0 / 3.0
_EPS = 1e-5
_TAU_INV = 2.0
_DECAY = 1e-4
_CL_W = 0.1
_DEG = 8


def _dot_t(a, x):
    return jax.lax.dot_general(
        a, x, (((0,), (0,)), ((), ())), preferred_element_type=F32)


def _simmat(x, y):
    return jax.lax.dot_general(
        x, y, (((1,), (1,)), ((), ())), preferred_element_type=F32)


def _body(a1_ref, a2_ref, t0_ref, te_ref, ce_ref, a3_ref,
          idx_ref, g_ref, b_ref, w1_ref, cb1_ref, w2_ref, cb2_ref,
          rel_ref, fcw_ref, fcb_ref,
          out_ref, aux_ref,
          u1_ref, u2_ref, st1_ref, st2_ref, sel_ref, ac1_ref, ac2_ref,
          *, nb, blk, ne, nt, nc, nr, ns, e_cnt):
    step = pl.program_id(0)

    @pl.when(step == 0)
    def _():
        u1_ref[...] = jnp.zeros_like(u1_ref)
        u2_ref[...] = jnp.zeros_like(u2_ref)
        sel_ref[...] = jnp.zeros_like(sel_ref)

    @pl.when(step < nb)
    def _sweep():
        a1 = a1_ref[...].astype(BF16)
        a2 = a2_ref[...].astype(BF16)
        t0f = t0_ref[...]
        t0 = t0f.astype(BF16)
        t1a = jnp.dot(a1, te_ref[...].astype(BF16),
                      preferred_element_type=F32)
        t1b = jnp.dot(a2, ce_ref[...].astype(BF16),
                      preferred_element_type=F32)
        r1 = jnp.concatenate([t0, t1a.astype(BF16)], axis=1)
        r2 = jnp.concatenate([t0, t1b.astype(BF16)], axis=1)
        u1_ref[...] += _dot_t(a1, r1)
        u2_ref[...] += _dot_t(a2, r2)
        base = pl.multiple_of(step * blk, blk)
        st1_ref[pl.ds(base, blk), :] = ((t0f + t1a)
                                        * _L_INV).astype(BF16)
        st2_ref[pl.ds(base, blk), :] = ((t0f + t1b)
                                        * _L_INV).astype(BF16)
        ac1_ref[pl.ds(base, blk), :] = a1
        ac2_ref[pl.ds(base, blk), :] = a2

    nb2 = max(1, nb // 4)
    blk2 = (nb * blk) // nb2

    @pl.when((step >= nb) & (step < nb + nb2))
    def _select():
        d = te_ref.shape[1]
        j = step - nb
        base = pl.multiple_of(j * blk2, blk2)
        v1 = (u1_ref[:, :d] * _L_INV).astype(BF16)
        v2 = (u2_ref[:, :d] * _L_INV).astype(BF16)
        a1 = ac1_ref[pl.ds(base, blk2), :]
        a2 = ac2_ref[pl.ds(base, blk2), :]
        pre1 = (st1_ref[pl.ds(base, blk2), :].astype(F32)
                + jnp.dot(a1, v1, preferred_element_type=F32))
        pre2 = (st2_ref[pl.ds(base, blk2), :].astype(F32)
                + jnp.dot(a2, v2, preferred_element_type=F32))
        ids = idx_ref[0:ns, :]
        iota_b = jax.lax.broadcasted_iota(jnp.int32, (ns, blk2), 1)
        oh = (iota_b == ids - base).astype(BF16)
        pre = jnp.concatenate([pre1, pre2], axis=1).astype(BF16)
        sel_ref[...] += jnp.dot(oh, pre, preferred_element_type=F32)

    @pl.when(step == nb + nb2)
    def _finish():
        g = g_ref[...]
        b = b_ref[...]

        def ln(x):
            mu = jnp.mean(x, axis=-1, keepdims=True)
            var = jnp.mean((x - mu) * (x - mu), axis=-1, keepdims=True)
            return (x - mu) * jax.lax.rsqrt(var + _EPS) * g + b

        te = te_ref[...]
        ce = ce_ref[...]
        d = te.shape[1]

        a3 = a3_ref[...].astype(BF16)
        b1_3 = _dot_t(a3, te.astype(BF16))
        t1_3 = jnp.dot(a3, ce.astype(BF16), preferred_element_type=F32)
        b2_3 = _dot_t(a3, t1_3.astype(BF16))
        tsum_3 = jnp.dot(a3, (ce + b1_3).astype(BF16),
                         preferred_element_type=F32)
        t2c_t = ln((te + tsum_3) * _L_INV)
        t2c_c = ln((ce + b1_3 + b2_3) * _L_INV)

        u1 = u1_ref[...]
        u2 = u2_ref[...]
        e2t_t = ln((te + u1[:, :d] + u1[:, d:]) * _L_INV)
        e2c_c = ln((ce + u2[:, :d] + u2[:, d:]) * _L_INV)

        sel = sel_ref[...]
        top1 = ln(sel[:, :d])
        top2 = ln(sel[:, d:])

        ids = idx_ref[0:ns, :]
        is_ent = ids < ne
        iota_t = jax.lax.broadcasted_iota(jnp.int32, (ns, nt), 1)
        iota_c = jax.lax.broadcasted_iota(jnp.int32, (ns, nc), 1)
        oh_t = ((iota_t == ids - ne) & (ids >= ne)
                & (ids < ne + nt)).astype(BF16)
        oh_c = ((iota_c == ids - (ne + nt)) & (ids >= ne + nt)).astype(BF16)
        small1 = jnp.concatenate([e2t_t, t2c_t], axis=1).astype(BF16)
        small2 = jnp.concatenate([t2c_c, e2c_c], axis=1).astype(BF16)
        both = (jnp.dot(oh_t, small1, preferred_element_type=F32)
                + jnp.dot(oh_c, small2, preferred_element_type=F32))
        s1 = jnp.where(is_ent, top1, 0.0) + both[:, :d]
        s2 = jnp.where(is_ent, top2, 0.0) + both[:, d:]

        def fc(x):
            h = (jnp.dot(x, w1_ref[...], preferred_element_type=F32)
                 + cb1_ref[...])
            h = jnp.where(h > 0.0, h, jnp.exp(jnp.minimum(h, 0.0)) - 1.0)
            return (jnp.dot(h, w2_ref[...], preferred_element_type=F32)
                    + cb2_ref[...])

        def normalize(z):
            nrm = jnp.sqrt(jnp.sum(z * z, axis=-1, keepdims=True))
            return z / jnp.maximum(nrm, 1e-12)

        av = normalize(fc(s1))
        bv = normalize(fc(s2))
        ab = av.astype(BF16)
        bb = bv.astype(BF16)
        self_sim = math.exp(_TAU_INV)
        r1m = jnp.exp(_simmat(ab, ab) * _TAU_INV)
        btm = jnp.exp(_simmat(ab, bb) * _TAU_INV)
        r2m = jnp.exp(_simmat(bb, bb) * _TAU_INV)
        pos = jnp.sum(av * bv, axis=-1, keepdims=True) * _TAU_INV
        denom1 = (jnp.sum(r1m, axis=1, keepdims=True)
                  + jnp.sum(btm, axis=1, keepdims=True) - self_sim)
        denom2 = (jnp.sum(r2m, axis=0, keepdims=True)
                  + jnp.sum(btm, axis=0, keepdims=True) - self_sim)
        total = (jnp.sum(jnp.log(denom1)) + jnp.sum(jnp.log(denom2))
                 - 2.0 * jnp.sum(pos))
        contrast = total * (0.5 / float(ns))

        et = idx_ref[ns + e_cnt:ns + 2 * e_cnt, :]
        lane_r = jax.lax.broadcasted_iota(jnp.int32, (e_cnt, nr), 1)
        rmod = et - nr * (et // nr)
        sgn = jnp.where(et >= nr, -1.0, 1.0)
        ohr = jnp.where(lane_r == rmod, sgn, 0.0).astype(BF16)
        rel = jnp.dot(ohr, rel_ref[...].astype(BF16),
                      preferred_element_type=F32)

        es = idx_ref[ns:ns + e_cnt, :]
        lane_n = jax.lax.broadcasted_iota(jnp.int32, (e_cnt, ns), 1)
        ohs = (lane_n == es).astype(BF16)
        srcc = jnp.concatenate([s1, s2], axis=1).astype(BF16)
        sm = jnp.dot(ohs, srcc, preferred_element_type=F32)

        emb_reg = 0.5 * (jnp.sum(s1 * s1) + jnp.sum(s2 * s2)
                         + jnp.sum(rel * rel))
        emb_loss = _DECAY * emb_reg / float(e_cnt)
        aux = _CL_W * contrast + emb_loss

        msg = jnp.maximum(sm + rel, 0.0)
        two_d = msg.shape[1]
        pooled = jnp.mean(msg.reshape(e_cnt // _DEG, _DEG, two_d), axis=1)
        predict = (jnp.dot(pooled.astype(BF16), fcw_ref[...].astype(BF16),
                           preferred_element_type=F32) + fcb_ref[...])
        out_ref[...] = jax.nn.sigmoid(predict)
        aux_ref[...] = jnp.zeros(aux_ref.shape, F32) + aux


def kernel(entity_emb, type_emb, cluster_emb, relation, ln_gamma, ln_beta,
           cl_w1, cl_b1, cl_w2, cl_b2, fc_w, fc_b,
           g_e2t, g_t2c, g_e2c, src_ids, etype, edge_src):
    ne, d = entity_emb.shape
    nt = type_emb.shape[0]
    nc = cluster_emb.shape[0]
    nr = relation.shape[0]
    ns = src_ids.shape[0]
    e_cnt = etype.shape[0]
    n_types = fc_w.shape[1]
    blk = 2048 if ne % 2048 == 0 else ne
    nb = ne // blk

    body = functools.partial(_body, nb=nb, blk=blk, ne=ne, nt=nt, nc=nc,
                             nr=nr, ns=ns, e_cnt=e_cnt)
    idx = jnp.concatenate([src_ids.astype(jnp.int32),
                           edge_src.astype(jnp.int32),
                           etype.astype(jnp.int32)])[:, None]

    def adj_spec(w):
        return pl.BlockSpec((blk, w), lambda i: (jnp.minimum(i, nb - 1), 0))

    def t0_spec():
        return pl.BlockSpec((blk, d), lambda i: (jnp.minimum(i, nb - 1), 0))

    def const_spec(shape):
        n_ = len(shape)
        return pl.BlockSpec(shape, lambda i, _n=n_: (0,) * _n)

    out, aux = pl.pallas_call(
        body,
        grid=(nb + max(1, nb // 4) + 1,),
        in_specs=[
            adj_spec(nt), adj_spec(nc), t0_spec(),
            const_spec((nt, d)), const_spec((nc, d)), const_spec((nt, nc)),
            const_spec((ns + 2 * e_cnt, 1)),
            const_spec((1, d)), const_spec((1, d)),
            const_spec((d, d)), const_spec((1, d)),
            const_spec((d, d)), const_spec((1, d)),
            const_spec((nr, 2 * d)), const_spec((2 * d, n_types)),
            const_spec((1, n_types)),
        ],
        out_specs=(const_spec((e_cnt // _DEG, n_types)),
                   const_spec((1, 128))),
        out_shape=(jax.ShapeDtypeStruct((e_cnt // _DEG, n_types), F32),
                   jax.ShapeDtypeStruct((1, 128), F32)),
        scratch_shapes=[pltpu.VMEM((nt, 2 * d), F32),
                        pltpu.VMEM((nc, 2 * d), F32),
                        pltpu.VMEM((ne, d), BF16),
                        pltpu.VMEM((ne, d), BF16),
                        pltpu.VMEM((ns, 2 * d), F32),
                        pltpu.VMEM((ne, nt), BF16),
                        pltpu.VMEM((ne, nc), BF16)],
        compiler_params=pltpu.CompilerParams(
            dimension_semantics=("arbitrary",),
            vmem_limit_bytes=100 * 1024 * 1024,
        ),
    )(g_e2t, g_e2c, entity_emb, type_emb, cluster_emb, g_t2c,
      idx, ln_gamma, ln_beta,
      cl_w1, cl_b1, cl_w2, cl_b2,
      relation, fc_w, fc_b)

    return out, aux[0, 0]
